# Initial kernel scaffold; baseline (speedup 1.0000x reference)
import jax
import jax.numpy as jnp
from jax.experimental import pallas as pl


def kernel(query, key, value, index_pair, query_batch_cnt, key_batch_cnt, index_pair_batch, in_proj_weight, in_proj_bias, out_proj_weight, out_proj_bias):
    raise NotImplementedError("write your pallas kernel here")


# speedup vs baseline: 2.9466x; 2.9466x over previous
import functools

import jax
import jax.numpy as jnp
from jax import lax
from jax.experimental import pallas as pl
from jax.experimental.pallas import tpu as pltpu
from jax.experimental.pallas import tpu_sc as plsc

_H = 8
_LN = 16
_NC = 2
_NS = 16
_NW = _NC * _NS


def _mm_body(x_ref, w_ref, b_ref, o_ref):
    o_ref[...] = (
        jnp.dot(x_ref[...], w_ref[...], preferred_element_type=jnp.float32,
                precision=lax.Precision.HIGHEST)
        + b_ref[...]
    )


def _matmul_bias(x, wt, b, bm=1024):
    n, c = x.shape
    co = wt.shape[1]
    return pl.pallas_call(
        _mm_body,
        grid=(n // bm,),
        in_specs=[
            pl.BlockSpec((bm, c), lambda i: (i, 0)),
            pl.BlockSpec((c, co), lambda i: (0, 0)),
            pl.BlockSpec((1, co), lambda i: (0, 0)),
        ],
        out_specs=pl.BlockSpec((bm, co), lambda i: (i, 0)),
        out_shape=jax.ShapeDtypeStruct((n, co), jnp.float32),
    )(x, wt, b.reshape(1, co))


def _sc_attention(qp, kp, vp, gidx, maskf):
    n, c = qp.shape
    dh = c // _H
    nsub = dh // 16
    qpw = n // _NW

    mesh = plsc.VectorSubcoreMesh(core_axis_name="c", subcore_axis_name="s")

    @functools.partial(
        pl.kernel,
        out_type=(
            jax.ShapeDtypeStruct((n, c), jnp.float32),
            jax.ShapeDtypeStruct((n, _LN), jnp.float32),
        ),
        mesh=mesh,
        scratch_types=[
            pltpu.VMEM((qpw, _LN), jnp.int32),
            pltpu.VMEM((qpw, _LN), jnp.float32),
            pltpu.VMEM((c,), jnp.float32),
            pltpu.VMEM((_LN, c), jnp.float32),
            pltpu.VMEM((_LN, c), jnp.float32),
            pltpu.VMEM((c,), jnp.float32),
            pltpu.VMEM((_LN,), jnp.float32),
            pltpu.SemaphoreType.DMA,
        ],
        compiler_params=pltpu.CompilerParams(
            use_tc_tiling_on_sc=False, needs_layout_passes=False),
    )
    def attn(qp_hbm, kp_hbm, vp_hbm, gidx_hbm, maskf_hbm, out_hbm, wsum_hbm,
             idx_v, mask_v, q_v, k_v, v_v, o_v, ws_v, sem):
        wid = lax.axis_index("s") * _NC + lax.axis_index("c")
        base = wid * qpw
        pltpu.sync_copy(gidx_hbm.at[pl.ds(base, qpw)], idx_v)
        pltpu.sync_copy(maskf_hbm.at[pl.ds(base, qpw)], mask_v)

        rows16 = lax.iota(jnp.int32, 16)
        lane_ids = [jnp.full((16,), l, jnp.int32) for l in range(_LN)]

        def body(i, carry):
            row = base + i
            pltpu.sync_copy(qp_hbm.at[row], q_v)
            pltpu.async_copy(kp_hbm.at[idx_v.at[i]], k_v, sem).wait()
            pltpu.async_copy(vp_hbm.at[idx_v.at[i]], v_v, sem).wait()

            pad = mask_v[i, :] > 0.5
            wacc = jnp.zeros((16,), jnp.float32)
            weights = []
            for h in range(_H):
                acc = jnp.zeros((16,), jnp.float32)
                for j in range(nsub):
                    qv = q_v[pl.ds(h * dh + 16 * j, 16)]
                    for t in range(16):
                        col = jnp.full((16,), h * dh + 16 * j + t, jnp.int32)
                        kcol = plsc.load_gather(k_v, [rows16, col])
                        acc = acc + kcol * qv[lane_ids[t]]
                sv = jnp.where(pad, jnp.float32(-1000.0), acc)
                e = jnp.exp(sv - jnp.max(sv))
                w = e / jnp.sum(e)
                weights.append(w)
                wacc = wacc + w
            ws_v[...] = wacc * jnp.float32(1.0 / _H)

            for h in range(_H):
                accs = [jnp.zeros((16,), jnp.float32) for _ in range(nsub)]
                for l in range(_LN):
                    wl = weights[h][lane_ids[l]]
                    for j in range(nsub):
                        accs[j] = accs[j] + wl * v_v[l, pl.ds(h * dh + 16 * j, 16)]
                for j in range(nsub):
                    o_v[pl.ds(h * dh + 16 * j, 16)] = accs[j]

            pltpu.sync_copy(o_v, out_hbm.at[row])
            pltpu.sync_copy(ws_v, wsum_hbm.at[row])
            return carry

        lax.fori_loop(0, qpw, body, 0)

    return attn(qp, kp, vp, gidx, maskf)


def kernel(query, key, value, index_pair, query_batch_cnt, key_batch_cnt,
           index_pair_batch, in_proj_weight, in_proj_bias, out_proj_weight,
           out_proj_bias):
    n, c = query.shape
    dh = c // _H
    scaling = float(dh) ** -0.5

    wq_t = in_proj_weight[:c].T * scaling
    wk_t = in_proj_weight[c:2 * c].T
    wv_t = in_proj_weight[2 * c:].T
    bq = in_proj_bias[:c] * scaling
    bk = in_proj_bias[c:2 * c]
    bv = in_proj_bias[2 * c:]

    mask = index_pair < 0
    key_start = jnp.concatenate([
        jnp.zeros((1,), jnp.int32),
        jnp.cumsum(key_batch_cnt)[:-1].astype(jnp.int32),
    ])
    offs = key_start[index_pair_batch]
    gidx = jnp.where(mask, 0, index_pair + offs[:, None])
    maskf = mask.astype(jnp.float32)

    qp = _matmul_bias(query, wq_t, bq)
    kp = _matmul_bias(key, wk_t, bk)
    vp = _matmul_bias(value, wv_t, bv)

    attn_out, wsum = _sc_attention(qp, kp, vp, gidx, maskf)

    out = _matmul_bias(attn_out, out_proj_weight.T, out_proj_bias)
    return out, wsum


# speedup vs baseline: 4.6244x; 1.5694x over previous
import functools

import jax
import jax.numpy as jnp
from jax import lax
from jax.experimental import pallas as pl
from jax.experimental.pallas import tpu as pltpu
from jax.experimental.pallas import tpu_sc as plsc

_H = 8
_LN = 16
_NC = 2
_NS = 16
_NW = _NC * _NS
_QT = 32


def _mm_body(x_ref, w_ref, b_ref, o_ref):
    o_ref[...] = (
        jnp.dot(x_ref[...], w_ref[...], preferred_element_type=jnp.float32,
                precision=lax.Precision.HIGHEST)
        + b_ref[...]
    )


def _matmul_bias(x, wt, b, bm=1024):
    n, c = x.shape
    co = wt.shape[1]
    return pl.pallas_call(
        _mm_body,
        grid=(n // bm,),
        in_specs=[
            pl.BlockSpec((bm, c), lambda i: (i, 0)),
            pl.BlockSpec((c, co), lambda i: (0, 0)),
            pl.BlockSpec((1, co), lambda i: (0, 0)),
        ],
        out_specs=pl.BlockSpec((bm, co), lambda i: (i, 0)),
        out_shape=jax.ShapeDtypeStruct((n, co), jnp.float32),
    )(x, wt, b.reshape(1, co))


def _kv_body(k_ref, v_ref, wk_ref, wv_ref, bk_ref, bv_ref, o_ref):
    c = k_ref.shape[1]
    o_ref[:, :c] = (
        jnp.dot(k_ref[...], wk_ref[...], preferred_element_type=jnp.float32,
                precision=lax.Precision.HIGHEST)
        + bk_ref[...]
    )
    o_ref[:, c:] = (
        jnp.dot(v_ref[...], wv_ref[...], preferred_element_type=jnp.float32,
                precision=lax.Precision.HIGHEST)
        + bv_ref[...]
    )


def _kv_proj(key, value, wk_t, wv_t, bk, bv, bm=1024):
    m, c = key.shape
    return pl.pallas_call(
        _kv_body,
        grid=(m // bm,),
        in_specs=[
            pl.BlockSpec((bm, c), lambda i: (i, 0)),
            pl.BlockSpec((bm, c), lambda i: (i, 0)),
            pl.BlockSpec((c, c), lambda i: (0, 0)),
            pl.BlockSpec((c, c), lambda i: (0, 0)),
            pl.BlockSpec((1, c), lambda i: (0, 0)),
            pl.BlockSpec((1, c), lambda i: (0, 0)),
        ],
        out_specs=pl.BlockSpec((bm, 2 * c), lambda i: (i, 0)),
        out_shape=jax.ShapeDtypeStruct((m, 2 * c), jnp.float32),
    )(key, value, wk_t, wv_t, bk.reshape(1, c), bv.reshape(1, c))


def _sc_attention(qp, kvp, gidx, maskf):
    n, c = qp.shape
    dh = c // _H
    nsub = dh // 16
    qpw = n // _NW

    mesh = plsc.VectorSubcoreMesh(core_axis_name="c", subcore_axis_name="s")

    @functools.partial(
        pl.kernel,
        out_type=(
            jax.ShapeDtypeStruct((n, c), jnp.float32),
            jax.ShapeDtypeStruct((n, _LN), jnp.float32),
        ),
        mesh=mesh,
        scratch_types=[
            pltpu.VMEM((qpw, _LN), jnp.int32),
            pltpu.VMEM((qpw, _LN), jnp.float32),
            pltpu.VMEM((_QT, c), jnp.float32),
            pltpu.VMEM((_LN, 2 * c), jnp.float32),
            pltpu.VMEM((_LN, 2 * c), jnp.float32),
            pltpu.VMEM((_QT, c), jnp.float32),
            pltpu.VMEM((_QT, _LN), jnp.float32),
            pltpu.SemaphoreType.DMA,
            pltpu.SemaphoreType.DMA,
        ],
        compiler_params=pltpu.CompilerParams(
            use_tc_tiling_on_sc=False, needs_layout_passes=False),
    )
    def attn(qp_hbm, kvp_hbm, gidx_hbm, maskf_hbm, out_hbm, wsum_hbm,
             idx_v, mask_v, q_v, kv_a, kv_b, o_v, ws_v, sem_a, sem_b):
        wid = lax.axis_index("s") * _NC + lax.axis_index("c")
        base = wid * qpw
        pltpu.sync_copy(gidx_hbm.at[pl.ds(base, qpw)], idx_v)
        pltpu.sync_copy(maskf_hbm.at[pl.ds(base, qpw)], mask_v)

        rows16 = lax.iota(jnp.int32, 16)
        lane_ids = [jnp.full((16,), l, jnp.int32) for l in range(_LN)]
        inv_h = jnp.float32(1.0 / _H)

        pltpu.async_copy(kvp_hbm.at[idx_v.at[0]], kv_a, sem_a)

        def process(i, kv_buf, kv_nbuf, sem, sem_n):
            qi = lax.rem(i, _QT)
            ip1 = i + 1

            @pl.when(qi == 0)
            def _():
                pltpu.sync_copy(qp_hbm.at[pl.ds(base + i, _QT)], q_v)

            @pl.when(ip1 < qpw)
            def _():
                pltpu.async_copy(kvp_hbm.at[idx_v.at[ip1]], kv_nbuf, sem_n)

            pltpu.make_async_copy(kvp_hbm.at[idx_v.at[i]], kv_buf, sem).wait()

            pad = mask_v[i, :] > 0.5

            def head(h, wacc):
                hoff = h * dh
                acc = jnp.zeros((16,), jnp.float32)
                for j in range(nsub):
                    qv = q_v[qi, pl.ds(hoff + 16 * j, 16)]
                    for t in range(16):
                        col = jnp.full((16,), hoff + 16 * j + t, jnp.int32)
                        kcol = plsc.load_gather(kv_buf, [rows16, col])
                        acc = acc + kcol * qv[lane_ids[t]]
                sv = jnp.where(pad, jnp.float32(-1000.0), acc)
                e = jnp.exp(sv - jnp.max(sv))
                w = e / jnp.sum(e)
                vaccs = [jnp.zeros((16,), jnp.float32) for _ in range(nsub)]
                for l in range(_LN):
                    wl = w[lane_ids[l]]
                    for j in range(nsub):
                        vaccs[j] = vaccs[j] + wl * kv_buf[l, pl.ds(c + hoff + 16 * j, 16)]
                for j in range(nsub):
                    o_v[qi, pl.ds(hoff + 16 * j, 16)] = vaccs[j]
                return wacc + w

            wacc = lax.fori_loop(0, _H, head, jnp.zeros((16,), jnp.float32))
            ws_v[qi, :] = wacc * inv_h

            @pl.when(qi == _QT - 1)
            def _():
                pltpu.sync_copy(o_v, out_hbm.at[pl.ds(base + i - (_QT - 1), _QT)])
                pltpu.sync_copy(ws_v, wsum_hbm.at[pl.ds(base + i - (_QT - 1), _QT)])

        def pair(g, carry):
            process(2 * g, kv_a, kv_b, sem_a, sem_b)
            process(2 * g + 1, kv_b, kv_a, sem_b, sem_a)
            return carry

        lax.fori_loop(0, qpw // 2, pair, 0)

    return attn(qp, kvp, gidx, maskf)


def kernel(query, key, value, index_pair, query_batch_cnt, key_batch_cnt,
           index_pair_batch, in_proj_weight, in_proj_bias, out_proj_weight,
           out_proj_bias):
    n, c = query.shape
    dh = c // _H
    scaling = float(dh) ** -0.5

    wq_t = in_proj_weight[:c].T * scaling
    wk_t = in_proj_weight[c:2 * c].T
    wv_t = in_proj_weight[2 * c:].T
    bq = in_proj_bias[:c] * scaling
    bk = in_proj_bias[c:2 * c]
    bv = in_proj_bias[2 * c:]

    mask = index_pair < 0
    key_start = jnp.concatenate([
        jnp.zeros((1,), jnp.int32),
        jnp.cumsum(key_batch_cnt)[:-1].astype(jnp.int32),
    ])
    offs = key_start[index_pair_batch]
    gidx = jnp.where(mask, 0, index_pair + offs[:, None])
    maskf = mask.astype(jnp.float32)

    qp = _matmul_bias(query, wq_t, bq)
    kvp = _kv_proj(key, value, wk_t, wv_t, bk, bv)

    attn_out, wsum = _sc_attention(qp, kvp, gidx, maskf)

    out = _matmul_bias(attn_out, out_proj_weight.T, out_proj_bias)
    return out, wsum


# speedup vs baseline: 4.9799x; 1.0769x over previous
import functools

import jax
import jax.numpy as jnp
from jax import lax
from jax.experimental import pallas as pl
from jax.experimental.pallas import tpu as pltpu
from jax.experimental.pallas import tpu_sc as plsc

_H = 8
_LN = 16
_NC = 2
_NS = 16
_NW = _NC * _NS
_QT = 32


def _mm_body(x_ref, w_ref, b_ref, o_ref):
    o_ref[...] = (
        jnp.dot(x_ref[...], w_ref[...], preferred_element_type=jnp.float32,
                precision=lax.Precision.HIGHEST)
        + b_ref[...]
    )


def _matmul_bias(x, wt, b, bm=1024):
    n, c = x.shape
    co = wt.shape[1]
    return pl.pallas_call(
        _mm_body,
        grid=(n // bm,),
        in_specs=[
            pl.BlockSpec((bm, c), lambda i: (i, 0)),
            pl.BlockSpec((c, co), lambda i: (0, 0)),
            pl.BlockSpec((1, co), lambda i: (0, 0)),
        ],
        out_specs=pl.BlockSpec((bm, co), lambda i: (i, 0)),
        out_shape=jax.ShapeDtypeStruct((n, co), jnp.float32),
    )(x, wt, b.reshape(1, co))


def _kv_body(k_ref, v_ref, wk_ref, wv_ref, bk_ref, bv_ref, o_ref):
    c = k_ref.shape[1]
    o_ref[:, :c] = (
        jnp.dot(k_ref[...], wk_ref[...], preferred_element_type=jnp.float32,
                precision=lax.Precision.HIGHEST)
        + bk_ref[...]
    )
    o_ref[:, c:] = (
        jnp.dot(v_ref[...], wv_ref[...], preferred_element_type=jnp.float32,
                precision=lax.Precision.HIGHEST)
        + bv_ref[...]
    )


def _kv_proj(key, value, wk_t, wv_t, bk, bv, bm=1024):
    m, c = key.shape
    return pl.pallas_call(
        _kv_body,
        grid=(m // bm,),
        in_specs=[
            pl.BlockSpec((bm, c), lambda i: (i, 0)),
            pl.BlockSpec((bm, c), lambda i: (i, 0)),
            pl.BlockSpec((c, c), lambda i: (0, 0)),
            pl.BlockSpec((c, c), lambda i: (0, 0)),
            pl.BlockSpec((1, c), lambda i: (0, 0)),
            pl.BlockSpec((1, c), lambda i: (0, 0)),
        ],
        out_specs=pl.BlockSpec((bm, 2 * c), lambda i: (i, 0)),
        out_shape=jax.ShapeDtypeStruct((m, 2 * c), jnp.float32),
    )(key, value, wk_t, wv_t, bk.reshape(1, c), bv.reshape(1, c))


def _sc_attention(qp, kvp, gidx, maskf):
    n, c = qp.shape
    dh = c // _H
    nsub = dh // 16
    qpw = n // _NW

    mesh = plsc.VectorSubcoreMesh(core_axis_name="c", subcore_axis_name="s")

    @functools.partial(
        pl.kernel,
        out_type=(
            jax.ShapeDtypeStruct((n, c), jnp.float32),
            jax.ShapeDtypeStruct((n, _LN), jnp.float32),
        ),
        mesh=mesh,
        scratch_types=[
            pltpu.VMEM((qpw, _LN), jnp.int32),
            pltpu.VMEM((qpw, _LN), jnp.float32),
            pltpu.VMEM((_QT, c), jnp.float32),
            pltpu.VMEM((_LN, 2 * c), jnp.float32),
            pltpu.VMEM((_LN, 2 * c), jnp.float32),
            pltpu.VMEM((_QT, c), jnp.float32),
            pltpu.VMEM((_QT, _LN), jnp.float32),
            pltpu.SemaphoreType.DMA,
            pltpu.SemaphoreType.DMA,
        ],
        compiler_params=pltpu.CompilerParams(
            use_tc_tiling_on_sc=False, needs_layout_passes=False),
    )
    def attn(qp_hbm, kvp_hbm, gidx_hbm, maskf_hbm, out_hbm, wsum_hbm,
             idx_v, mask_v, q_v, kv_a, kv_b, o_v, ws_v, sem_a, sem_b):
        wid = lax.axis_index("s") * _NC + lax.axis_index("c")
        base = wid * qpw
        pltpu.sync_copy(gidx_hbm.at[pl.ds(base, qpw)], idx_v)
        pltpu.sync_copy(maskf_hbm.at[pl.ds(base, qpw)], mask_v)

        rows16 = lax.iota(jnp.int32, 16)
        lane_ids = [jnp.full((16,), l, jnp.int32) for l in range(_LN)]
        inv_h = jnp.float32(1.0 / _H)

        pltpu.async_copy(kvp_hbm.at[idx_v.at[0]], kv_a, sem_a)

        def process(i, kv_buf, kv_nbuf, sem, sem_n):
            qi = lax.rem(i, _QT)
            ip1 = i + 1

            @pl.when(qi == 0)
            def _():
                pltpu.sync_copy(qp_hbm.at[pl.ds(base + i, _QT)], q_v)

            @pl.when(ip1 < qpw)
            def _():
                pltpu.async_copy(kvp_hbm.at[idx_v.at[ip1]], kv_nbuf, sem_n)

            pltpu.make_async_copy(kvp_hbm.at[idx_v.at[i]], kv_buf, sem).wait()

            pad = mask_v[i, :] > 0.5

            def one_head(h):
                hoff = h * dh
                paccs = []
                for j in range(nsub):
                    qv = q_v[qi, pl.ds(hoff + 16 * j, 16)]
                    a0 = jnp.zeros((16,), jnp.float32)
                    a1 = jnp.zeros((16,), jnp.float32)
                    for t in range(16):
                        col = jnp.full((16,), hoff + 16 * j + t, jnp.int32)
                        term = plsc.load_gather(kv_buf, [rows16, col]) * qv[lane_ids[t]]
                        if t % 2 == 0:
                            a0 = a0 + term
                        else:
                            a1 = a1 + term
                    paccs.append(a0 + a1)
                acc = (paccs[0] + paccs[1]) + (paccs[2] + paccs[3])
                sv = jnp.where(pad, jnp.float32(-1000.0), acc)
                e = jnp.exp(sv - jnp.max(sv))
                w = e / jnp.sum(e)
                va = [jnp.zeros((16,), jnp.float32) for _ in range(nsub)]
                vb = [jnp.zeros((16,), jnp.float32) for _ in range(nsub)]
                for l in range(_LN):
                    wl = w[lane_ids[l]]
                    dst = va if l % 2 == 0 else vb
                    for j in range(nsub):
                        dst[j] = dst[j] + wl * kv_buf[l, pl.ds(c + hoff + 16 * j, 16)]
                for j in range(nsub):
                    o_v[qi, pl.ds(hoff + 16 * j, 16)] = va[j] + vb[j]
                return w

            def head2(g, wacc):
                w0 = one_head(2 * g)
                w1 = one_head(2 * g + 1)
                return wacc + w0 + w1

            wacc = lax.fori_loop(0, _H // 2, head2,
                                 jnp.zeros((16,), jnp.float32))
            ws_v[qi, :] = wacc * inv_h

            @pl.when(qi == _QT - 1)
            def _():
                pltpu.sync_copy(o_v, out_hbm.at[pl.ds(base + i - (_QT - 1), _QT)])
                pltpu.sync_copy(ws_v, wsum_hbm.at[pl.ds(base + i - (_QT - 1), _QT)])

        def pair(g, carry):
            process(2 * g, kv_a, kv_b, sem_a, sem_b)
            process(2 * g + 1, kv_b, kv_a, sem_b, sem_a)
            return carry

        lax.fori_loop(0, qpw // 2, pair, 0)

    return attn(qp, kvp, gidx, maskf)


def kernel(query, key, value, index_pair, query_batch_cnt, key_batch_cnt,
           index_pair_batch, in_proj_weight, in_proj_bias, out_proj_weight,
           out_proj_bias):
    n, c = query.shape
    dh = c // _H
    scaling = float(dh) ** -0.5

    wq_t = in_proj_weight[:c].T * scaling
    wk_t = in_proj_weight[c:2 * c].T
    wv_t = in_proj_weight[2 * c:].T
    bq = in_proj_bias[:c] * scaling
    bk = in_proj_bias[c:2 * c]
    bv = in_proj_bias[2 * c:]

    mask = index_pair < 0
    key_start = jnp.concatenate([
        jnp.zeros((1,), jnp.int32),
        jnp.cumsum(key_batch_cnt)[:-1].astype(jnp.int32),
    ])
    offs = key_start[index_pair_batch]
    gidx = jnp.where(mask, 0, index_pair + offs[:, None])
    maskf = mask.astype(jnp.float32)

    qp = _matmul_bias(query, wq_t, bq)
    kvp = _kv_proj(key, value, wk_t, wv_t, bk, bv)

    attn_out, wsum = _sc_attention(qp, kvp, gidx, maskf)

    out = _matmul_bias(attn_out, out_proj_weight.T, out_proj_bias)
    return out, wsum
